# Initial kernel scaffold; baseline (speedup 1.0000x reference)
import jax
import jax.numpy as jnp
from jax.experimental import pallas as pl


def kernel(bot_mlp_input, cat_features, bw0, bb0, bw1, bb1, bw2, bb2, emb, tw0, tb0, tw1, tb1, tw2, tb2, tw3, tb3, tw4, tb4):
    raise NotImplementedError("write your pallas kernel here")


# speedup vs baseline: 10.7115x; 10.7115x over previous
import functools
import numpy as np
import jax
import jax.numpy as jnp
from jax import lax
from jax.experimental import pallas as pl
from jax.experimental.pallas import tpu as pltpu
from jax.experimental.pallas import tpu_sc as plsc

B = 4096
NS = 26
D = 128
NF = 27
NIDX = B * NS
VOCAB = 100000


_NC = 2
_NSUB = 16
_NW = _NC * _NSUB
_PER_W = NIDX // _NW
_CHUNK = 128
_NCHUNK = _PER_W // _CHUNK


def _sc_gather_body(idx_hbm, emb_hbm, out_hbm, idx_v, buf0, buf1, sem0, sem1):
  wid = lax.axis_index("s") * _NC + lax.axis_index("c")
  base = wid * _PER_W
  pltpu.sync_copy(idx_hbm.at[pl.ds(base, _PER_W)], idx_v)

  def start(c, buf, sem):
    return pltpu.async_copy(emb_hbm.at[idx_v.at[pl.ds(c * _CHUNK, _CHUNK)]],
                            buf, sem)

  start(0, buf0, sem0)

  def body(c, carry):
    @pl.when(c % 2 == 0)
    def _():
      @pl.when(c + 1 < _NCHUNK)
      def _():
        start(c + 1, buf1, sem1)
      pltpu.make_async_copy(emb_hbm.at[idx_v.at[pl.ds(c * _CHUNK, _CHUNK)]],
                            buf0, sem0).wait()
      pltpu.sync_copy(buf0, out_hbm.at[pl.ds(base + c * _CHUNK, _CHUNK)])

    @pl.when(c % 2 == 1)
    def _():
      @pl.when(c + 1 < _NCHUNK)
      def _():
        start(c + 1, buf0, sem0)
      pltpu.make_async_copy(emb_hbm.at[idx_v.at[pl.ds(c * _CHUNK, _CHUNK)]],
                            buf1, sem1).wait()
      pltpu.sync_copy(buf1, out_hbm.at[pl.ds(base + c * _CHUNK, _CHUNK)])
    return carry

  lax.fori_loop(0, _NCHUNK, body, 0)


def _sc_gather(idx, emb):
  mesh = plsc.VectorSubcoreMesh(core_axis_name="c", subcore_axis_name="s")
  f = pl.kernel(
      _sc_gather_body,
      mesh=mesh,
      out_type=jax.ShapeDtypeStruct((NIDX, D), jnp.float32),
      scratch_types=[
          pltpu.VMEM((_PER_W,), jnp.int32),
          pltpu.VMEM((_CHUNK, D), jnp.float32),
          pltpu.VMEM((_CHUNK, D), jnp.float32),
          pltpu.SemaphoreType.DMA,
          pltpu.SemaphoreType.DMA,
      ],
  )
  return f(idx, emb)


_BT = 256


def _tc_body(x_ref, embf_ref, bw0_ref, bb0_ref, bw1_ref, bb1_ref, bw2_ref,
             bb2_ref, t0b_ref, wpair_ref, tb0_ref, tw1_ref, tb1_ref, tw2_ref,
             tb2_ref, tw3_ref, tb3_ref, tw4_ref, tb4_ref, out_ref):
  x = x_ref[...]
  h = jnp.maximum(jnp.dot(x, bw0_ref[...],
                          preferred_element_type=jnp.float32) + bb0_ref[...], 0.0)
  h = jnp.maximum(jnp.dot(h, bw1_ref[...],
                          preferred_element_type=jnp.float32) + bb1_ref[...], 0.0)
  bot = jnp.maximum(jnp.dot(h, bw2_ref[...],
                            preferred_element_type=jnp.float32) + bb2_ref[...], 0.0)

  emb3 = embf_ref[...].reshape(_BT, NS, D)
  feat = jnp.concatenate([bot.reshape(_BT, 1, D), emb3], axis=1)
  xact = lax.dot_general(feat, feat,
                         dimension_numbers=(((2,), (2,)), ((0,), (0,))),
                         preferred_element_type=jnp.float32)

  h = jnp.dot(xact.reshape(_BT, NF * NF), wpair_ref[...],
              preferred_element_type=jnp.float32)
  h = h + jnp.dot(bot, t0b_ref[...], preferred_element_type=jnp.float32)
  h = jnp.maximum(h + tb0_ref[...], 0.0)
  h = jnp.maximum(jnp.dot(h, tw1_ref[...],
                          preferred_element_type=jnp.float32) + tb1_ref[...], 0.0)
  h = jnp.maximum(jnp.dot(h, tw2_ref[...],
                          preferred_element_type=jnp.float32) + tb2_ref[...], 0.0)
  h = jnp.maximum(jnp.dot(h, tw3_ref[...],
                          preferred_element_type=jnp.float32) + tb3_ref[...], 0.0)
  out_ref[...] = jnp.dot(h, tw4_ref[...],
                         preferred_element_type=jnp.float32) + tb4_ref[...]


def _const(shape):
  nd = len(shape)
  return pl.BlockSpec(shape, lambda i: (0,) * nd)


def _tc_forward(xp, embf, bw0p, bb0, bw1, bb1, bw2, bb2, t0b, wpair, tb0, tw1,
                tb1, tw2, tb2, tw3, tb3, tw4p, tb4p):
  grid = (B // _BT,)
  return pl.pallas_call(
      _tc_body,
      grid=grid,
      in_specs=[
          pl.BlockSpec((_BT, 16), lambda i: (i, 0)),
          pl.BlockSpec((_BT, NS * D), lambda i: (i, 0)),
          _const((16, 512)),
          _const((1, 512)),
          _const((512, 256)),
          _const((1, 256)),
          _const((256, 128)),
          _const((1, 128)),
          _const((128, 1024)),
          _const((NF * NF, 1024)),
          _const((1, 1024)),
          _const((1024, 1024)),
          _const((1, 1024)),
          _const((1024, 512)),
          _const((1, 512)),
          _const((512, 256)),
          _const((1, 256)),
          _const((256, 128)),
          _const((1, 128)),
      ],
      out_specs=pl.BlockSpec((_BT, 128), lambda i: (i, 0)),
      out_shape=jax.ShapeDtypeStruct((B, 128), jnp.float32),
  )(xp, embf, bw0p, bb0, bw1, bb1, bw2, bb2, t0b, wpair, tb0, tw1, tb1, tw2,
    tb2, tw3, tb3, tw4p, tb4p)


def kernel(bot_mlp_input, cat_features, bw0, bb0, bw1, bb1, bw2, bb2, emb,
           tw0, tb0, tw1, tb1, tw2, tb2, tw3, tb3, tw4, tb4):
  offsets = jnp.arange(NS, dtype=jnp.int32) * VOCAB
  idx = (cat_features.astype(jnp.int32) + offsets[None, :]).reshape(-1)

  embf = _sc_gather(idx, emb).reshape(B, NS * D)

  xp = jnp.pad(bot_mlp_input, ((0, 0), (0, 3)))
  bw0p = jnp.pad(bw0, ((0, 3), (0, 0)))

  iu = np.triu_indices(NF)
  pmat = np.zeros((NF, NF), dtype=np.int32)
  pmat[iu] = np.arange(NF * (NF + 1) // 2, dtype=np.int32)
  pmat = pmat + pmat.T - np.diag(np.diag(pmat))
  scale = np.full((NF, NF, 1), 0.5, dtype=np.float32)
  scale[np.arange(NF), np.arange(NF), 0] = 1.0
  t0b = tw0[:D]
  wpair = tw0[D:][pmat.reshape(-1)].reshape(NF, NF, 1024) * scale
  wpair = wpair.reshape(NF * NF, 1024)

  tw4p = jnp.pad(tw4, ((0, 0), (0, 127)))
  tb4p = jnp.pad(tb4.reshape(1, 1), ((0, 0), (0, 127)))

  out = _tc_forward(xp, embf, bw0p, bb0.reshape(1, -1), bw1,
                    bb1.reshape(1, -1), bw2, bb2.reshape(1, -1), t0b, wpair,
                    tb0.reshape(1, -1), tw1, tb1.reshape(1, -1), tw2,
                    tb2.reshape(1, -1), tw3, tb3.reshape(1, -1), tw4p, tb4p)
  return out[:, :1]
